# baseline (speedup 1.0000x reference)
import functools

import jax
import jax.numpy as jnp
from jax import lax
from jax.experimental import pallas as pl
from jax.experimental.pallas import tpu as pltpu
from jax.experimental.pallas import tpu_sc as plsc

B, N, M, NS, C = 8, 4096, 1024, 32, 128
RADIUS = 0.12
R2 = RADIUS * RADIUS

NC, NSUB, L = 2, 16, 16
MQ = M // NSUB
NG = MQ // L
KC = 8
TROWS = 136
NSLAB = 17
OUTC = 3 + C
MNS = M * NS
POSQ = MNS // 4
SUBP = 2048
NSUBB = POSQ // SUBP
BPC = B // NC

_BCAST_DNUMS = lax.GatherDimensionNumbers(
    offset_dims=(), collapsed_slice_dims=(0,), start_index_map=(0,))


def _bcast(vec, j):
    return lax.gather(vec, jnp.full((L, 1), j, jnp.int32), _BCAST_DNUMS, (1,),
                      mode=lax.GatherScatterMode.PROMISE_IN_BOUNDS)


def _ball_query(pxyz_v, q_v, idx_v, lanes):
    zeros = jnp.zeros((L,), jnp.int32)
    qs = []
    for grp in range(NG):
        qsel = (grp * L + lanes) * 3
        qs.append((plsc.load_gather(q_v, [zeros, qsel]),
                   plsc.load_gather(q_v, [zeros, qsel + 1]),
                   plsc.load_gather(q_v, [zeros, qsel + 2]),
                   (grp * L + lanes) * NS))
        plsc.store_scatter(idx_v, [qs[grp][3]], zeros)

    def step(k, cnts):
        base = k * L
        basev = jnp.full((L,), base, jnp.int32)
        pxc = pxyz_v[0, pl.ds(base, L)]
        pyc = pxyz_v[1, pl.ds(base, L)]
        pzc = pxyz_v[2, pl.ds(base, L)]
        new = list(cnts)
        for j in range(L):
            px = _bcast(pxc, j)
            py = _bcast(pyc, j)
            pz = _bcast(pzc, j)
            pvec = basev + j
            for g in range(NG):
                qx, qy, qz, qoff = qs[g]
                cnt = new[g]
                dx = qx - px
                dy = qy - py
                dz = qz - pz
                d2 = dx * dx + dy * dy + dz * dz
                mask = d2 < R2
                wmask = mask & (cnt < NS)
                plsc.store_scatter(idx_v, [qoff + cnt], pvec, mask=wmask)
                new[g] = cnt + mask.astype(jnp.int32)
        return tuple(new)

    cnts = lax.fori_loop(0, N // L, step, (jnp.zeros((L,), jnp.int32),) * NG)

    for g in range(NG):
        qoff = qs[g][3]
        cnt = cnts[g]
        first = plsc.load_gather(idx_v, [qoff])
        for s in range(1, NS):
            cur = plsc.load_gather(idx_v, [qoff + s])
            sel = jnp.where(cnt > s, cur, first)
            plsc.store_scatter(idx_v, [qoff + s], sel)


def _body(tab_hbm, q_hbm, q4_hbm, out_hbm,
          pxyz_v, q_v, q4_v, idx_v, iq_v, frows_v, obuf_v, idx_sh, sems):
    ci = lax.axis_index("c")
    sid = lax.axis_index("s")
    lanes = lax.iota(jnp.int32, L)
    mc = lax.shift_right_logical(sid, 2)
    cc = sid & 3
    pbase = pl.multiple_of(mc * POSQ, 128)

    def per_batch(bi, _):
        b = ci * BPC + bi
        with jax.named_scope("bq_phase"):
            pltpu.sync_copy(tab_hbm.at[b, pl.ds(0, 3), :], pxyz_v)
            pltpu.sync_copy(q_hbm.at[b, sid], q_v)
            _ball_query(pxyz_v, q_v, idx_v, lanes)
            pltpu.sync_copy(idx_v, idx_sh.at[pl.ds(sid * MQ * NS, MQ * NS)])
        with jax.named_scope("bar1"):
            plsc.subcore_barrier()

        with jax.named_scope("gather_phase"):
            pltpu.sync_copy(idx_sh.at[pl.ds(pbase, POSQ)], iq_v)
            pltpu.sync_copy(q4_hbm.at[b, mc], q4_v)

            def slab_of(k):
                return pl.multiple_of((cc + 4 * k) * KC, KC)

            def fin(k, slot):
                return pltpu.make_async_copy(
                    tab_hbm.at[b, pl.ds(slab_of(k), KC), :], frows_v.at[slot],
                    sems.at[slot])

            def fout(k, sub, slot, nrow, row0):
                return pltpu.make_async_copy(
                    obuf_v.at[slot, pl.ds(0, nrow)],
                    out_hbm.at[b, pl.ds(row0, nrow),
                               pl.ds(pbase + sub * SUBP, SUBP)],
                    sems.at[2 + slot])

            def gather_block(sub, slot, fslot, fixup):
                def gath(t, _):
                    iv = iq_v[pl.ds(sub * SUBP + t * L, L)]
                    for c in range(KC):
                        g = plsc.load_gather(
                            frows_v, [jnp.full((L,), fslot, jnp.int32),
                                      jnp.full((L,), c, jnp.int32), iv])
                        if fixup and c < 3:
                            mv3 = lax.shift_right_logical(
                                sub * SUBP + t * L + lanes, 5) * 3
                            qd = plsc.load_gather(
                                q4_v, [jnp.zeros((L,), jnp.int32), mv3 + c])
                            g = g - qd
                        obuf_v[slot, c, pl.ds(t * L, L)] = g
                    return 0
                lax.fori_loop(0, SUBP // L, gath, 0)

            fin(0, 0).start()
            for k in range(NSLAB // 4):
                fslot = k % 2
                fin(k, fslot).wait()
                if k + 1 < NSLAB // 4:
                    fin(k + 1, 1 - fslot).start()
                row0 = slab_of(k)
                for sub in range(NSUBB):
                    g = k * NSUBB + sub
                    slot = g % 2
                    if g >= 2:
                        fout(*divmod(g - 2, NSUBB), slot, KC,
                             slab_of((g - 2) // NSUBB)).wait()
                    if k == 0:
                        @pl.when(cc == 0)
                        def _():
                            gather_block(sub, slot, fslot, True)
                        @pl.when(cc != 0)
                        def _():
                            gather_block(sub, slot, fslot, False)
                    else:
                        gather_block(sub, slot, fslot, False)
                    fout(k, sub, slot, KC, row0).start()
            for g in (14, 15):
                fout(*divmod(g, NSUBB), g % 2, KC, slab_of(g // NSUBB)).wait()

            @pl.when(cc == 1)
            def _():
                pltpu.sync_copy(tab_hbm.at[b, pl.ds(16 * KC, KC), :],
                                frows_v.at[0])
                for sub in range(NSUBB):
                    slot = sub % 2
                    if sub >= 2:
                        fout(4, sub - 2, slot, 3, 16 * KC).wait()
                    gather_block(sub, slot, 0, False)
                    fout(4, sub, slot, 3, 16 * KC).start()
                for sub in (NSUBB - 2, NSUBB - 1):
                    fout(4, sub, sub % 2, 3, 16 * KC).wait()

        with jax.named_scope("bar2"):
            plsc.subcore_barrier()
        return 0

    lax.fori_loop(0, BPC, per_batch, 0)


@functools.partial(
    pl.kernel,
    out_type=jax.ShapeDtypeStruct((B, OUTC, MNS), jnp.float32),
    mesh=plsc.VectorSubcoreMesh(core_axis_name="c", subcore_axis_name="s"),
    scratch_types=[
        pltpu.VMEM((3, N), jnp.float32),
        pltpu.VMEM((1, MQ * 3), jnp.float32),
        pltpu.VMEM((1, (M // 4) * 3), jnp.float32),
        pltpu.VMEM((MQ * NS,), jnp.int32),
        pltpu.VMEM((POSQ,), jnp.int32),
        pltpu.VMEM((2, KC, N), jnp.float32),
        pltpu.VMEM((2, KC, SUBP), jnp.float32),
        pltpu.VMEM_SHARED((MNS,), jnp.int32),
        pltpu.SemaphoreType.DMA((4,)),
    ],
    compiler_params=pltpu.CompilerParams(needs_layout_passes=False),
)
def _qg_kernel(tab_hbm, q_hbm, q4_hbm, out_hbm, *scratch):
    _body(tab_hbm, q_hbm, q4_hbm, out_hbm, *scratch)


def kernel(xyz, new_xyz, features):
    xyz_t = jnp.transpose(xyz, (0, 2, 1))
    pad = jnp.zeros((B, TROWS - 3 - C, N), jnp.float32)
    tab = jnp.concatenate([xyz_t, features, pad], axis=1)
    q = new_xyz.reshape(B, NSUB, 1, MQ * 3)
    q4 = new_xyz.reshape(B, 4, 1, (M // 4) * 3)
    out = _qg_kernel(tab, q, q4)
    return out.reshape(B, OUTC, M, NS)


# speedup vs baseline: 1.0865x; 1.0865x over previous
import functools

import jax
import jax.numpy as jnp
from jax import lax
from jax.experimental import pallas as pl
from jax.experimental.pallas import tpu as pltpu
from jax.experimental.pallas import tpu_sc as plsc

B, N, M, NS, C = 8, 4096, 1024, 32, 128
RADIUS = 0.12
R2 = RADIUS * RADIUS

NC, NSUB, L = 2, 16, 16
MQ = M // NSUB
NG = MQ // L
KC = 8
TROWS = 136
NSLAB = 17
OUTC = 3 + C
MNS = M * NS
POSQ = MNS // 4
SUBP = 2048
NSUBB = POSQ // SUBP
BPC = B // NC

_BCAST_DNUMS = lax.GatherDimensionNumbers(
    offset_dims=(), collapsed_slice_dims=(0,), start_index_map=(0,))


def _bcast(vec, j):
    return lax.gather(vec, jnp.full((L, 1), j, jnp.int32), _BCAST_DNUMS, (1,),
                      mode=lax.GatherScatterMode.PROMISE_IN_BOUNDS)


def _ball_query(pxyz_v, q_v, idx_v, lanes):
    zeros = jnp.zeros((L,), jnp.int32)
    qs = []
    for grp in range(NG):
        qsel = (grp * L + lanes) * 3
        qs.append((plsc.load_gather(q_v, [zeros, qsel]),
                   plsc.load_gather(q_v, [zeros, qsel + 1]),
                   plsc.load_gather(q_v, [zeros, qsel + 2]),
                   (grp * L + lanes) * NS))
        plsc.store_scatter(idx_v, [qs[grp][3]], zeros)
    qlims = [qs[g][3] + NS for g in range(NG)]

    def step(k, slots):
        base = k * L
        basev = jnp.full((L,), base, jnp.int32)
        pxc = pxyz_v[0, pl.ds(base, L)]
        pyc = pxyz_v[1, pl.ds(base, L)]
        pzc = pxyz_v[2, pl.ds(base, L)]
        new = list(slots)
        for j in range(L):
            px = _bcast(pxc, j)
            py = _bcast(pyc, j)
            pz = _bcast(pzc, j)
            pvec = basev + j
            for g in range(NG):
                qx, qy, qz, qoff = qs[g]
                slot = new[g]
                dx = qx - px
                dy = qy - py
                dz = qz - pz
                d2 = dx * dx + dy * dy + dz * dz
                mask = d2 < R2
                wmask = mask & (slot < qlims[g])
                plsc.store_scatter(idx_v, [slot], pvec, mask=wmask)
                new[g] = slot + mask.astype(jnp.int32)
        return tuple(new)

    slots = lax.fori_loop(0, N // L, step,
                          tuple(qs[g][3] for g in range(NG)))

    for g in range(NG):
        qoff = qs[g][3]
        cnt = slots[g] - qoff
        first = plsc.load_gather(idx_v, [qoff])
        for s in range(1, NS):
            cur = plsc.load_gather(idx_v, [qoff + s])
            sel = jnp.where(cnt > s, cur, first)
            plsc.store_scatter(idx_v, [qoff + s], sel)


def _body(tab_hbm, q_hbm, q4_hbm, out_hbm,
          pxyz_v, q_v, q4_v, idx_v, iq_v, frows_v, obuf_v, idx_sh, sems):
    ci = lax.axis_index("c")
    sid = lax.axis_index("s")
    lanes = lax.iota(jnp.int32, L)
    mc = lax.shift_right_logical(sid, 2)
    cc = sid & 3
    pbase = pl.multiple_of(mc * POSQ, 128)

    def per_batch(bi, _):
        b = ci * BPC + bi
        with jax.named_scope("bq_phase"):
            pltpu.sync_copy(tab_hbm.at[b, pl.ds(0, 3), :], pxyz_v)
            pltpu.sync_copy(q_hbm.at[b, sid], q_v)
            _ball_query(pxyz_v, q_v, idx_v, lanes)
            pltpu.sync_copy(idx_v, idx_sh.at[pl.ds(sid * MQ * NS, MQ * NS)])
        with jax.named_scope("bar1"):
            plsc.subcore_barrier()

        with jax.named_scope("gather_phase"):
            pltpu.sync_copy(idx_sh.at[pl.ds(pbase, POSQ)], iq_v)
            pltpu.sync_copy(q4_hbm.at[b, mc], q4_v)

            def slab_of(k):
                return pl.multiple_of((cc + 4 * k) * KC, KC)

            def fin(k, slot):
                return pltpu.make_async_copy(
                    tab_hbm.at[b, pl.ds(slab_of(k), KC), :], frows_v.at[slot],
                    sems.at[slot])

            def fout(k, sub, slot, nrow, row0):
                return pltpu.make_async_copy(
                    obuf_v.at[slot, pl.ds(0, nrow)],
                    out_hbm.at[b, pl.ds(row0, nrow),
                               pl.ds(pbase + sub * SUBP, SUBP)],
                    sems.at[2 + slot])

            def gather_block(sub, slot, fslot, fixup, nch=KC):
                def gath(t, _):
                    iv = iq_v[pl.ds(sub * SUBP + t * L, L)]
                    for c in range(nch):
                        g = plsc.load_gather(
                            frows_v, [jnp.full((L,), fslot, jnp.int32),
                                      jnp.full((L,), c, jnp.int32), iv])
                        if fixup and c < 3:
                            mv3 = lax.shift_right_logical(
                                sub * SUBP + t * L + lanes, 5) * 3
                            qd = plsc.load_gather(
                                q4_v, [jnp.zeros((L,), jnp.int32), mv3 + c])
                            g = g - qd
                        obuf_v[slot, c, pl.ds(t * L, L)] = g
                    return 0
                lax.fori_loop(0, SUBP // L, gath, 0)

            fin(0, 0).start()
            for k in range(NSLAB // 4):
                fslot = k % 2
                fin(k, fslot).wait()
                if k + 1 < NSLAB // 4:
                    fin(k + 1, 1 - fslot).start()
                row0 = slab_of(k)
                for sub in range(NSUBB):
                    g = k * NSUBB + sub
                    slot = g % 2
                    if g >= 2:
                        fout(*divmod(g - 2, NSUBB), slot, KC,
                             slab_of((g - 2) // NSUBB)).wait()
                    if k == 0:
                        @pl.when(cc == 0)
                        def _():
                            gather_block(sub, slot, fslot, True)
                        @pl.when(cc != 0)
                        def _():
                            gather_block(sub, slot, fslot, False)
                    else:
                        gather_block(sub, slot, fslot, False)
                    fout(k, sub, slot, KC, row0).start()
            for g in (14, 15):
                fout(*divmod(g, NSUBB), g % 2, KC, slab_of(g // NSUBB)).wait()

            pltpu.sync_copy(tab_hbm.at[b, pl.ds(16 * KC, 3), :],
                            frows_v.at[0, pl.ds(0, 3)])

            def tgath(t, _):
                iv = iq_v[pl.ds(pl.multiple_of(cc * SUBP, 16) + t * L, L)]
                for c in range(3):
                    g = plsc.load_gather(
                        frows_v, [jnp.zeros((L,), jnp.int32),
                                  jnp.full((L,), c, jnp.int32), iv])
                    obuf_v[0, c, pl.ds(t * L, L)] = g
                return 0

            lax.fori_loop(0, SUBP // L, tgath, 0)
            pltpu.sync_copy(
                obuf_v.at[0, pl.ds(0, 3)],
                out_hbm.at[b, pl.ds(16 * KC, 3),
                           pl.ds(pbase + pl.multiple_of(cc * SUBP, 128),
                                 SUBP)])

        with jax.named_scope("bar2"):
            plsc.subcore_barrier()
        return 0

    lax.fori_loop(0, BPC, per_batch, 0)


@functools.partial(
    pl.kernel,
    out_type=jax.ShapeDtypeStruct((B, OUTC, MNS), jnp.float32),
    mesh=plsc.VectorSubcoreMesh(core_axis_name="c", subcore_axis_name="s"),
    scratch_types=[
        pltpu.VMEM((3, N), jnp.float32),
        pltpu.VMEM((1, MQ * 3), jnp.float32),
        pltpu.VMEM((1, (M // 4) * 3), jnp.float32),
        pltpu.VMEM((MQ * NS,), jnp.int32),
        pltpu.VMEM((POSQ,), jnp.int32),
        pltpu.VMEM((2, KC, N), jnp.float32),
        pltpu.VMEM((2, KC, SUBP), jnp.float32),
        pltpu.VMEM_SHARED((MNS,), jnp.int32),
        pltpu.SemaphoreType.DMA((4,)),
    ],
    compiler_params=pltpu.CompilerParams(needs_layout_passes=False),
)
def _qg_kernel(tab_hbm, q_hbm, q4_hbm, out_hbm, *scratch):
    _body(tab_hbm, q_hbm, q4_hbm, out_hbm, *scratch)


def kernel(xyz, new_xyz, features):
    xyz_t = jnp.transpose(xyz, (0, 2, 1))
    pad = jnp.zeros((B, TROWS - 3 - C, N), jnp.float32)
    tab = jnp.concatenate([xyz_t, features, pad], axis=1)
    q = new_xyz.reshape(B, NSUB, 1, MQ * 3)
    q4 = new_xyz.reshape(B, 4, 1, (M // 4) * 3)
    out = _qg_kernel(tab, q, q4)
    return out.reshape(B, OUTC, M, NS)
